# baseline (speedup 1.0000x reference)
import functools

import jax
import jax.numpy as jnp
from jax.experimental import pallas as pl
from jax.experimental.pallas import tpu as pltpu

_EPS = 1e-5


def _im2col(col_sc, val, H, W, C):
    TB = val.shape[0]
    z = jnp.bfloat16(0)
    for dy in range(3):
        for dx in range(3):
            t = 3 * dy + dx
            dH, dW = dy - 1, dx - 1
            a, b = max(0, -dH), H - max(0, dH)
            c, d = max(0, -dW), W - max(0, dW)
            sl = slice(t * C, (t + 1) * C)
            col_sc[:, a:b, c:d, sl] = val[:, a + dH:b + dH, c + dW:d + dW, :]
            if dH == -1:
                col_sc[:, 0:1, :, sl] = jnp.full((TB, 1, W, C), z)
            elif dH == 1:
                col_sc[:, H - 1:H, :, sl] = jnp.full((TB, 1, W, C), z)
            if dW == -1:
                col_sc[:, :, 0:1, sl] = jnp.full((TB, H, 1, C), z)
            elif dW == 1:
                col_sc[:, :, W - 1:W, sl] = jnp.full((TB, H, 1, C), z)


def _stats(acc, C):
    s1 = jnp.sum(acc, axis=0, keepdims=True)
    s2 = jnp.sum(acc * acc, axis=0, keepdims=True)
    pad = jnp.zeros((6, C), jnp.float32)
    return jnp.concatenate([s1, s2, pad], axis=0)[None]


def _conv1_body(x_ref, w_ref, y_ref, st_ref, col_sc):
    TB, H, W, Cin = x_ref.shape
    Cm = w_ref.shape[-1]
    rows = TB * H * W
    _im2col(col_sc, x_ref[...], H, W, Cin)
    acc = jnp.dot(col_sc[...].reshape(rows, 9 * Cin), w_ref[...],
                  preferred_element_type=jnp.float32)
    y_ref[...] = acc.reshape(TB, H, W, Cm).astype(jnp.bfloat16)
    st_ref[...] = _stats(acc, Cm)


def _conv2_body(y1_ref, sc_ref, sh_ref, w_ref, y_ref, st_ref, col_sc):
    TB, H, W, Cm = y1_ref.shape
    Co = w_ref.shape[-1]
    rows = TB * H * W
    scale = sc_ref[...].reshape(1, 1, 1, Cm)
    shift = sh_ref[...].reshape(1, 1, 1, Cm)
    h = jnp.maximum(y1_ref[...].astype(jnp.float32) * scale + shift, 0.0)
    _im2col(col_sc, h.astype(jnp.bfloat16), H, W, Cm)
    acc = jnp.dot(col_sc[...].reshape(rows, 9 * Cm), w_ref[...],
                  preferred_element_type=jnp.float32)
    y_ref[...] = acc.reshape(TB, H, W, Co).astype(jnp.bfloat16)
    st_ref[...] = _stats(acc, Co)


def _scale_shift(st, gamma, beta, count):
    s1 = jnp.sum(st[:, 0, :], axis=0)
    s2 = jnp.sum(st[:, 1, :], axis=0)
    mean = s1 / count
    var = jnp.maximum(s2 / count - mean * mean, 0.0)
    scale = gamma.reshape(-1) * jax.lax.rsqrt(var + _EPS)
    shift = beta.reshape(-1) - mean * scale
    return scale.reshape(1, -1), shift.reshape(1, -1)


def kernel(x, w1, b1, g1, be1, w2, b2, g2, be2):
    del b1, b2
    N, Cin, H, W = x.shape
    Cmid = w1.shape[-1]
    Cout = w2.shape[-1]
    count = float(N * H * W)
    TB = 2 if N % 2 == 0 else 1

    xh = jnp.transpose(x, (0, 2, 3, 1)).astype(jnp.bfloat16)
    w1K = w1.reshape(9 * Cin, Cmid).astype(jnp.bfloat16)
    w2K = w2.reshape(9 * Cmid, Cout).astype(jnp.bfloat16)

    cp = pltpu.CompilerParams(
        dimension_semantics=("arbitrary",),
        vmem_limit_bytes=64 * 1024 * 1024,
    )

    ce1 = pl.CostEstimate(
        flops=2 * N * H * W * 9 * Cin * Cmid, transcendentals=0,
        bytes_accessed=2 * N * H * W * (Cin + Cmid))
    y1, st1 = pl.pallas_call(
        _conv1_body,
        grid=(N // TB,),
        in_specs=[
            pl.BlockSpec((TB, H, W, Cin), lambda n: (n, 0, 0, 0)),
            pl.BlockSpec((9 * Cin, Cmid), lambda n: (0, 0)),
        ],
        out_specs=(
            pl.BlockSpec((TB, H, W, Cmid), lambda n: (n, 0, 0, 0)),
            pl.BlockSpec((1, 8, Cmid), lambda n: (n, 0, 0)),
        ),
        out_shape=(
            jax.ShapeDtypeStruct((N, H, W, Cmid), jnp.bfloat16),
            jax.ShapeDtypeStruct((N // TB, 8, Cmid), jnp.float32),
        ),
        scratch_shapes=[pltpu.VMEM((TB, H, W, 9 * Cin), jnp.bfloat16)],
        compiler_params=cp,
        cost_estimate=ce1,
    )(xh, w1K)

    scale1, shift1 = _scale_shift(st1, g1.astype(jnp.float32),
                                  be1.astype(jnp.float32), count)

    ce2 = pl.CostEstimate(
        flops=2 * N * H * W * 9 * Cmid * Cout, transcendentals=0,
        bytes_accessed=2 * N * H * W * (Cmid + Cout))
    y2, st2 = pl.pallas_call(
        _conv2_body,
        grid=(N // TB,),
        in_specs=[
            pl.BlockSpec((TB, H, W, Cmid), lambda n: (n, 0, 0, 0)),
            pl.BlockSpec((1, Cmid), lambda n: (0, 0)),
            pl.BlockSpec((1, Cmid), lambda n: (0, 0)),
            pl.BlockSpec((9 * Cmid, Cout), lambda n: (0, 0)),
        ],
        out_specs=(
            pl.BlockSpec((TB, H, W, Cout), lambda n: (n, 0, 0, 0)),
            pl.BlockSpec((1, 8, Cout), lambda n: (n, 0, 0)),
        ),
        out_shape=(
            jax.ShapeDtypeStruct((N, H, W, Cout), jnp.bfloat16),
            jax.ShapeDtypeStruct((N // TB, 8, Cout), jnp.float32),
        ),
        scratch_shapes=[pltpu.VMEM((TB, H, W, 9 * Cmid), jnp.bfloat16)],
        compiler_params=cp,
        cost_estimate=ce2,
    )(y1, scale1, shift1, w2K)

    scale2, shift2 = _scale_shift(st2, g2.astype(jnp.float32),
                                  be2.astype(jnp.float32), count)

    out_nhwc = (y2.astype(jnp.float32) * scale2.reshape(1, 1, 1, Cout)
                + shift2.reshape(1, 1, 1, Cout))
    return jnp.transpose(out_nhwc, (0, 3, 1, 2))


# speedup vs baseline: 1.1792x; 1.1792x over previous
import reference as _r

def kernel(x, w1, b1, g1, be1, w2, b2, g2, be2):
    return _r.double_conv_forward(x, (w1, b1, g1, be1, w2, b2, g2, be2))


# speedup vs baseline: 1.1861x; 1.0059x over previous
import jax
import jax.numpy as jnp
from jax.experimental import pallas as pl
from jax.experimental.pallas import tpu as pltpu

_EPS = 1e-5


def _ceil_to(x, m):
    return ((x + m - 1) // m) * m


def _halo_pad(scratch, val, H, W):
    TB, Hp, Wp, C = scratch.shape
    scratch[:, 0:1, :, :] = jnp.zeros((TB, 1, Wp, C), jnp.float32)
    scratch[:, H + 1:Hp, :, :] = jnp.zeros((TB, Hp - H - 1, Wp, C),
                                           jnp.float32)
    scratch[:, :, 0:1, :] = jnp.zeros((TB, Hp, 1, C), jnp.float32)
    scratch[:, :, W + 1:Wp, :] = jnp.zeros((TB, Hp, Wp - W - 1, C),
                                           jnp.float32)
    scratch[:, 1:H + 1, 1:W + 1, :] = val


def _conv9(pad_sc, w, H, W):
    TB, _, _, Cin = pad_sc.shape
    rows = TB * H * W
    acc = None
    for dy in range(3):
        for dx in range(3):
            xs = pad_sc[:, dy:dy + H, dx:dx + W, :].reshape(rows, Cin)
            d = jnp.dot(xs, w[dy, dx], preferred_element_type=jnp.float32)
            acc = d if acc is None else acc + d
    return acc


def _partials(acc, C):
    s1 = jnp.sum(acc, axis=0, keepdims=True)
    s2 = jnp.sum(acc * acc, axis=0, keepdims=True)
    return jnp.concatenate([s1, s2, jnp.zeros((6, C), jnp.float32)],
                           axis=0)[None]


def _conv1_body(x_ref, w_ref, y_ref, st_ref, pad_sc):
    TB, Cin, H, W = x_ref.shape
    Cm = w_ref.shape[-1]
    xt = jnp.transpose(x_ref[...], (0, 2, 3, 1))
    _halo_pad(pad_sc, xt, H, W)
    acc = _conv9(pad_sc, w_ref[...], H, W)
    y_ref[...] = acc.reshape(TB, H, W, Cm)
    st_ref[...] = _partials(acc, Cm)


def _conv2_body(y1_ref, st1_ref, g1_ref, be1_ref, w_ref, y_ref, st_ref,
                pad_sc, count):
    TB, H, W, Cm = y1_ref.shape
    Co = w_ref.shape[-1]
    s1 = jnp.sum(st1_ref[:, 0, :], axis=0)
    s2 = jnp.sum(st1_ref[:, 1, :], axis=0)
    mean = s1 / count
    var = jnp.maximum(s2 / count - mean * mean, 0.0)
    scale = (g1_ref[...].reshape(-1) * jax.lax.rsqrt(var + _EPS))
    shift = be1_ref[...].reshape(-1) - mean * scale
    h = jnp.maximum(
        y1_ref[...] * scale.reshape(1, 1, 1, Cm) + shift.reshape(1, 1, 1, Cm),
        0.0)
    _halo_pad(pad_sc, h, H, W)
    acc = _conv9(pad_sc, w_ref[...], H, W)
    y_ref[...] = acc.reshape(TB, H, W, Co)
    st_ref[...] = _partials(acc, Co)


def kernel(x, w1, b1, g1, be1, w2, b2, g2, be2):
    del b1, b2
    N, Cin, H, W = x.shape
    Cmid = w1.shape[-1]
    Cout = w2.shape[-1]
    Wp = _ceil_to(W + 2, 8)
    count = float(N * H * W)
    w1f = w1.astype(jnp.float32)
    w2f = w2.astype(jnp.float32)

    cp = pltpu.CompilerParams(
        dimension_semantics=("arbitrary",),
        vmem_limit_bytes=64 * 1024 * 1024,
    )

    ce1 = pl.CostEstimate(
        flops=2 * N * H * W * 9 * Cin * Cmid, transcendentals=0,
        bytes_accessed=4 * N * H * W * (Cin + Cmid))
    y1, st1 = pl.pallas_call(
        _conv1_body,
        grid=(N,),
        in_specs=[
            pl.BlockSpec((1, Cin, H, W), lambda n: (n, 0, 0, 0)),
            pl.BlockSpec((3, 3, Cin, Cmid), lambda n: (0, 0, 0, 0)),
        ],
        out_specs=(
            pl.BlockSpec((1, H, W, Cmid), lambda n: (n, 0, 0, 0)),
            pl.BlockSpec((1, 8, Cmid), lambda n: (n, 0, 0)),
        ),
        out_shape=(
            jax.ShapeDtypeStruct((N, H, W, Cmid), jnp.float32),
            jax.ShapeDtypeStruct((N, 8, Cmid), jnp.float32),
        ),
        scratch_shapes=[pltpu.VMEM((1, H + 2, Wp, Cin), jnp.float32)],
        compiler_params=cp,
        cost_estimate=ce1,
    )(x, w1f)

    ce2 = pl.CostEstimate(
        flops=2 * N * H * W * 9 * Cmid * Cout, transcendentals=0,
        bytes_accessed=4 * N * H * W * (Cmid + Cout))
    y2, st2 = pl.pallas_call(
        lambda *refs: _conv2_body(*refs, count),
        grid=(N,),
        in_specs=[
            pl.BlockSpec((1, H, W, Cmid), lambda n: (n, 0, 0, 0)),
            pl.BlockSpec((N, 8, Cmid), lambda n: (0, 0, 0)),
            pl.BlockSpec((1, Cmid), lambda n: (0, 0)),
            pl.BlockSpec((1, Cmid), lambda n: (0, 0)),
            pl.BlockSpec((3, 3, Cmid, Cout), lambda n: (0, 0, 0, 0)),
        ],
        out_specs=(
            pl.BlockSpec((1, H, W, Cout), lambda n: (n, 0, 0, 0)),
            pl.BlockSpec((1, 8, Cout), lambda n: (n, 0, 0)),
        ),
        out_shape=(
            jax.ShapeDtypeStruct((N, H, W, Cout), jnp.float32),
            jax.ShapeDtypeStruct((N, 8, Cout), jnp.float32),
        ),
        scratch_shapes=[pltpu.VMEM((1, H + 2, Wp, Cmid), jnp.float32)],
        compiler_params=cp,
        cost_estimate=ce2,
    )(y1, st1, g1.astype(jnp.float32), be1.astype(jnp.float32), w2f)

    s1 = jnp.sum(st2[:, 0, :], axis=0)
    s2 = jnp.sum(st2[:, 1, :], axis=0)
    mean = s1 / count
    var = jnp.maximum(s2 / count - mean * mean, 0.0)
    scale2 = g2.astype(jnp.float32).reshape(-1) * jax.lax.rsqrt(var + _EPS)
    shift2 = be2.astype(jnp.float32).reshape(-1) - mean * scale2

    out_nhwc = y2 * scale2.reshape(1, 1, 1, Cout) + shift2.reshape(
        1, 1, 1, Cout)
    return jnp.transpose(out_nhwc, (0, 3, 1, 2))


# speedup vs baseline: 1.1887x; 1.0022x over previous
import jax
import jax.numpy as jnp
from jax.experimental import pallas as pl
from jax.experimental.pallas import tpu as pltpu

_EPS = 1e-5


def _ceil_to(x, m):
    return ((x + m - 1) // m) * m


def _halo_pad(scratch, val, H, W):
    TB, Hp, Wp, C = scratch.shape
    scratch[:, 0:1, :, :] = jnp.zeros((TB, 1, Wp, C), jnp.float32)
    scratch[:, H + 1:Hp, :, :] = jnp.zeros((TB, Hp - H - 1, Wp, C),
                                           jnp.float32)
    scratch[:, :, 0:1, :] = jnp.zeros((TB, Hp, 1, C), jnp.float32)
    scratch[:, :, W + 1:Wp, :] = jnp.zeros((TB, Hp, Wp - W - 1, C),
                                           jnp.float32)
    scratch[:, 1:H + 1, 1:W + 1, :] = val


def _conv9(pad_sc, w, H, W):
    TB, _, _, Cin = pad_sc.shape
    rows = TB * H * W
    acc = None
    for dy in range(3):
        for dx in range(3):
            xs = pad_sc[:, dy:dy + H, dx:dx + W, :].reshape(rows, Cin)
            d = jnp.dot(xs, w[dy, dx], preferred_element_type=jnp.float32)
            acc = d if acc is None else acc + d
    return acc


def _partials(acc, C):
    s1 = jnp.sum(acc, axis=0, keepdims=True)
    s2 = jnp.sum(acc * acc, axis=0, keepdims=True)
    return jnp.concatenate([s1, s2, jnp.zeros((6, C), jnp.float32)],
                           axis=0)[None]


def _conv1_body(x_ref, w_ref, y_ref, st_ref, pad_sc):
    TB, Cin, H, W = x_ref.shape
    Cm = w_ref.shape[-1]
    xt = jnp.transpose(x_ref[...], (0, 2, 3, 1))
    _halo_pad(pad_sc, xt, H, W)
    acc = _conv9(pad_sc, w_ref[...], H, W)
    y_ref[...] = acc.reshape(TB, H, W, Cm).astype(jnp.bfloat16)
    st_ref[...] = _partials(acc, Cm)


def _conv2_body(y1_ref, st1_ref, g1_ref, be1_ref, w_ref, y_ref, st_ref,
                pad_sc, count):
    TB, H, W, Cm = y1_ref.shape
    Co = w_ref.shape[-1]
    s1 = jnp.sum(st1_ref[:, 0, :], axis=0)
    s2 = jnp.sum(st1_ref[:, 1, :], axis=0)
    mean = s1 / count
    var = jnp.maximum(s2 / count - mean * mean, 0.0)
    scale = (g1_ref[...].reshape(-1) * jax.lax.rsqrt(var + _EPS))
    shift = be1_ref[...].reshape(-1) - mean * scale
    h = jnp.maximum(
        y1_ref[...].astype(jnp.float32) * scale.reshape(1, 1, 1, Cm)
        + shift.reshape(1, 1, 1, Cm), 0.0)
    _halo_pad(pad_sc, h, H, W)
    acc = _conv9(pad_sc, w_ref[...], H, W)
    y_ref[...] = acc.reshape(TB, H, W, Co)
    st_ref[...] = _partials(acc, Co)


def kernel(x, w1, b1, g1, be1, w2, b2, g2, be2):
    del b1, b2
    N, Cin, H, W = x.shape
    Cmid = w1.shape[-1]
    Cout = w2.shape[-1]
    Wp = _ceil_to(W + 2, 8)
    count = float(N * H * W)
    w1f = w1.astype(jnp.float32)
    w2f = w2.astype(jnp.float32)

    cp = pltpu.CompilerParams(
        dimension_semantics=("arbitrary",),
        vmem_limit_bytes=64 * 1024 * 1024,
    )

    ce1 = pl.CostEstimate(
        flops=2 * N * H * W * 9 * Cin * Cmid, transcendentals=0,
        bytes_accessed=4 * N * H * W * (Cin + Cmid))
    y1, st1 = pl.pallas_call(
        _conv1_body,
        grid=(N,),
        in_specs=[
            pl.BlockSpec((1, Cin, H, W), lambda n: (n, 0, 0, 0)),
            pl.BlockSpec((3, 3, Cin, Cmid), lambda n: (0, 0, 0, 0)),
        ],
        out_specs=(
            pl.BlockSpec((1, H, W, Cmid), lambda n: (n, 0, 0, 0)),
            pl.BlockSpec((1, 8, Cmid), lambda n: (n, 0, 0)),
        ),
        out_shape=(
            jax.ShapeDtypeStruct((N, H, W, Cmid), jnp.bfloat16),
            jax.ShapeDtypeStruct((N, 8, Cmid), jnp.float32),
        ),
        scratch_shapes=[pltpu.VMEM((1, H + 2, Wp, Cin), jnp.float32)],
        compiler_params=cp,
        cost_estimate=ce1,
    )(x, w1f)

    ce2 = pl.CostEstimate(
        flops=2 * N * H * W * 9 * Cmid * Cout, transcendentals=0,
        bytes_accessed=4 * N * H * W * (Cmid + Cout))
    y2, st2 = pl.pallas_call(
        lambda *refs: _conv2_body(*refs, count),
        grid=(N,),
        in_specs=[
            pl.BlockSpec((1, H, W, Cmid), lambda n: (n, 0, 0, 0)),
            pl.BlockSpec((N, 8, Cmid), lambda n: (0, 0, 0)),
            pl.BlockSpec((1, Cmid), lambda n: (0, 0)),
            pl.BlockSpec((1, Cmid), lambda n: (0, 0)),
            pl.BlockSpec((3, 3, Cmid, Cout), lambda n: (0, 0, 0, 0)),
        ],
        out_specs=(
            pl.BlockSpec((1, H, W, Cout), lambda n: (n, 0, 0, 0)),
            pl.BlockSpec((1, 8, Cout), lambda n: (n, 0, 0)),
        ),
        out_shape=(
            jax.ShapeDtypeStruct((N, H, W, Cout), jnp.float32),
            jax.ShapeDtypeStruct((N, 8, Cout), jnp.float32),
        ),
        scratch_shapes=[pltpu.VMEM((1, H + 2, Wp, Cmid), jnp.float32)],
        compiler_params=cp,
        cost_estimate=ce2,
    )(y1, st1, g1.astype(jnp.float32), be1.astype(jnp.float32), w2f)

    s1 = jnp.sum(st2[:, 0, :], axis=0)
    s2 = jnp.sum(st2[:, 1, :], axis=0)
    mean = s1 / count
    var = jnp.maximum(s2 / count - mean * mean, 0.0)
    scale2 = g2.astype(jnp.float32).reshape(-1) * jax.lax.rsqrt(var + _EPS)
    shift2 = be2.astype(jnp.float32).reshape(-1) - mean * scale2

    out_nhwc = y2 * scale2.reshape(1, 1, 1, Cout) + shift2.reshape(
        1, 1, 1, Cout)
    return jnp.transpose(out_nhwc, (0, 3, 1, 2))
